# Initial kernel scaffold; baseline (speedup 1.0000x reference)
import jax
import jax.numpy as jnp
from jax.experimental import pallas as pl


def kernel(outputs, target):
    raise NotImplementedError("write your pallas kernel here")


# speedup vs baseline: 18.3659x; 18.3659x over previous
import functools

import jax
import jax.numpy as jnp
from jax import lax
from jax.experimental import pallas as pl
from jax.experimental.pallas import tpu as pltpu

THRESH = 0.7
MIN_KEPT = 100000
C = 19
B = 4
H = 512
W = 512
N = B * H * W
ROWS = 64


def _ce_block(out_ref, tgt_ref, loss_ref, pred_ref):
    x = out_ref[0]
    t = tgt_ref[0]
    m = jnp.max(x, axis=0)
    e = jnp.exp(x - m[None])
    s = jnp.sum(e, axis=0)
    cls = lax.broadcasted_iota(jnp.int32, (C, ROWS, W), 0)
    xt = jnp.sum(jnp.where(cls == t[None], x, 0.0), axis=0)
    z = xt - m
    loss_ref[0] = jnp.log(s) - z
    pred_ref[0] = jnp.exp(z) / s


def _ce_stage(outputs, target):
    grid = (B, H // ROWS)
    return pl.pallas_call(
        _ce_block,
        grid=grid,
        in_specs=[
            pl.BlockSpec((1, C, ROWS, W), lambda b, r: (b, 0, r, 0)),
            pl.BlockSpec((1, ROWS, W), lambda b, r: (b, r, 0)),
        ],
        out_specs=[
            pl.BlockSpec((1, ROWS, W), lambda b, r: (b, r, 0)),
            pl.BlockSpec((1, ROWS, W), lambda b, r: (b, r, 0)),
        ],
        out_shape=[
            jax.ShapeDtypeStruct((B, H, W), jnp.float32),
            jax.ShapeDtypeStruct((B, H, W), jnp.float32),
        ],
    )(outputs, target)


def _select_block(pred_ref, loss_ref, out_ref):
    bits = lax.bitcast_convert_type(pred_ref[...], jnp.int32)
    K = MIN_KEPT + 1

    def body(_, carry):
        lo, hi = carry
        mid = (lo + hi) // 2
        cnt = jnp.sum((bits <= mid).astype(jnp.int32))
        return jnp.where(cnt >= K, lo, mid), jnp.where(cnt >= K, mid, hi)

    lo0 = jnp.int32(-1)
    hi0 = jnp.int32(0x3F800001)
    lo, hi = lax.fori_loop(0, 31, body, (lo0, hi0))
    kth = lax.bitcast_convert_type(hi, jnp.float32)
    thresh = jnp.maximum(kth, THRESH)
    pred = pred_ref[...]
    keep = pred < thresh
    total = jnp.sum(jnp.where(keep, loss_ref[...], 0.0))
    denom = jnp.maximum(jnp.sum(keep.astype(jnp.float32)), 1.0)
    out_ref[...] = jnp.full((8, 128), total / denom, jnp.float32)


def _select_stage(pred, loss):
    return pl.pallas_call(
        _select_block,
        out_shape=jax.ShapeDtypeStruct((8, 128), jnp.float32),
    )(pred, loss)


@jax.jit
def kernel(outputs, target):
    loss, pred = _ce_stage(outputs, target)
    res = _select_stage(pred.reshape(1024, 1024), loss.reshape(1024, 1024))
    return res[0, 0]
